# baseline (speedup 1.0000x reference)
import functools

import jax
import jax.numpy as jnp
from jax import lax
from jax.experimental import pallas as pl
from jax.experimental.pallas import tpu as pltpu
from jax.experimental.pallas import tpu_sc as plsc

_NUM_WORKERS = 32
_LANES = 16
_NBUF = 4


def kernel(x, token_emb, pos_emb):
    B, S = x.shape
    V, D = token_emb.shape
    HALF = S // 2
    NBLK = 2 * (B // _NUM_WORKERS)

    x2 = x.reshape(B * 2, HALF).astype(jnp.int32)

    mesh = plsc.VectorSubcoreMesh(core_axis_name="c", subcore_axis_name="s")

    @functools.partial(
        pl.kernel,
        mesh=mesh,
        out_type=jax.ShapeDtypeStruct((B * 2, HALF, D), jnp.float32),
        scratch_types=[
            pltpu.VMEM((NBLK, HALF), jnp.int32),
            pltpu.VMEM((S, D), jnp.float32),
        ] + [pltpu.VMEM((HALF, D), jnp.float32) for _ in range(_NBUF)]
          + [pltpu.SemaphoreType.DMA for _ in range(2 * _NBUF)],
    )
    def run(tok_hbm, idx_hbm, pos_hbm, out_hbm, idx_v, pos_v, *rest):
        bufs = rest[:_NBUF]
        gsems = rest[_NBUF:2 * _NBUF]
        wsems = rest[2 * _NBUF:]
        wid = lax.axis_index("s") * 2 + lax.axis_index("c")
        blk_base = wid * NBLK
        pltpu.sync_copy(idx_hbm.at[pl.ds(blk_base, NBLK)], idx_v)
        pltpu.sync_copy(pos_hbm, pos_v)

        def issue_gather(blk, j):
            pltpu.async_copy(tok_hbm.at[idx_v.at[blk]], bufs[j], gsems[j])

        def wait_gather(blk, j):
            pltpu.make_async_copy(tok_hbm.at[idx_v.at[blk]], bufs[j],
                                  gsems[j]).wait()

        def issue_writeback(blk, j):
            pltpu.async_copy(bufs[j], out_hbm.at[blk_base + blk], wsems[j])

        def wait_writeback(j):
            pltpu.make_async_copy(bufs[j], out_hbm.at[0], wsems[j]).wait()

        def add_pos(j, phase):
            buf = bufs[j]

            @pl.loop(0, HALF)
            def _(r):
                for c in range(D // _LANES):
                    sl = pl.ds(c * _LANES, _LANES)
                    plsc.addupdate(buf.at[r, sl], pos_v[phase + r, sl])

        for j in range(_NBUF - 1):
            issue_gather(j, j)

        @pl.loop(0, NBLK // _NBUF)
        def _(t):
            for b in range(_NBUF):
                blk = _NBUF * t + b
                jg = (b + _NBUF - 1) % _NBUF
                blk_g = blk + _NBUF - 1

                if b == 0:
                    @pl.when(t > 0)
                    def _():
                        wait_writeback(jg)
                        issue_gather(blk_g, jg)

                    @pl.when(t == 0)
                    def _():
                        issue_gather(blk_g, jg)
                else:
                    @pl.when(blk_g < NBLK)
                    def _():
                        wait_writeback(jg)
                        issue_gather(blk_g, jg)

                wait_gather(blk, b)
                add_pos(b, (b % 2) * HALF)
                issue_writeback(blk, b)

        for j in range(_NBUF):
            wait_writeback(j)

    out = run(token_emb, x2, pos_emb)
    return out.reshape(B, S, D)
